# baseline (speedup 1.0000x reference)
import functools

import jax
import jax.numpy as jnp
from jax import lax
from jax.experimental import pallas as pl
from jax.experimental.pallas import tpu as pltpu
from jax.experimental.pallas import tpu_sc as plsc

_NC, _NS = 2, 16
_NW = _NC * _NS
_PC = 8
_UNROLL = 8


def _make_sc_kernel(B, L, D):
    lpw = L // _NW
    NP = lpw // _PC
    CW = _PC * D

    mesh = plsc.VectorSubcoreMesh(
        core_axis_name="c", subcore_axis_name="s", num_cores=_NC, num_subcores=_NS
    )

    scratch = (
        [pltpu.VMEM((CW,), jnp.float32) for _ in range(8)]
        + [pltpu.VMEM((CW,), jnp.float32) for _ in range(2)]
        + [pltpu.SemaphoreType.DMA for _ in range(18)]
    )

    def body(x_hbm, pos_hbm, out_hbm, *scr):
        xb = scr[0:8]
        pb = scr[8:10]
        sx = scr[10:18]
        so = scr[18:26]
        sp = scr[26:28]

        wid = lax.axis_index("s") * _NC + lax.axis_index("c")
        lbase = wid * lpw
        pos0 = lbase * D

        def start_pos(p, par):
            pltpu.async_copy(pos_hbm.at[pl.ds(pos0 + p * CW, CW)], pb[par], sp[par])

        def wait_pos(par):
            pltpu.make_async_copy(pos_hbm.at[pl.ds(0, CW)], pb[par], sp[par]).wait()

        def xoff(p, b):
            return (b * L + lbase + p * _PC) * D

        def start_x(p, b, par):
            s = par * 4 + b
            pltpu.async_copy(x_hbm.at[pl.ds(xoff(p, b), CW)], xb[s], sx[s])

        def wait_x(b, par):
            s = par * 4 + b
            pltpu.make_async_copy(x_hbm.at[pl.ds(0, CW)], xb[s], sx[s]).wait()

        def start_out(p, b, par):
            s = par * 4 + b
            pltpu.async_copy(xb[s], out_hbm.at[pl.ds(xoff(p, b), CW)], so[s])

        def wait_out(b, par):
            s = par * 4 + b
            pltpu.make_async_copy(xb[s], out_hbm.at[pl.ds(0, CW)], so[s]).wait()

        def add_chunk(b, par):
            pref = pb[par]
            xref = xb[par * 4 + b]
            span = 16 * _UNROLL

            def bf(j, _):
                base = j * span
                for u in range(_UNROLL):
                    off = base + u * 16
                    plsc.addupdate(xref.at[pl.ds(off, 16)], pref[pl.ds(off, 16)])
                return 0

            lax.fori_loop(0, CW // span, bf, 0)

        start_pos(0, 0)
        for b in range(B):
            start_x(0, b, 0)

        def loop_body(i, _):
            for par in range(2):
                p = 2 * i + par
                if par == 0:
                    start_pos(p + 1, 1)
                    for b in range(B):
                        @pl.when(i >= 1)
                        def _(b=b):
                            wait_out(b, 1)
                        start_x(p + 1, b, 1)
                else:
                    @pl.when(i < NP // 2 - 1)
                    def _():
                        start_pos(p + 1, 0)
                        for b in range(B):
                            wait_out(b, 0)
                            start_x(p + 1, b, 0)
                wait_pos(par)
                for b in range(B):
                    wait_x(b, par)
                    add_chunk(b, par)
                    start_out(p, b, par)
            return 0

        lax.fori_loop(0, NP // 2, loop_body, 0)

        for b in range(B):
            wait_out(b, 0)
            wait_out(b, 1)

    return mesh, scratch, body


def kernel(x, pos_table):
    B, L, D = x.shape
    mesh, scratch, body = _make_sc_kernel(B, L, D)
    xf = x.reshape(B * L * D)
    pf = pos_table[:L].reshape(L * D)
    out = pl.kernel(
        body,
        out_type=jax.ShapeDtypeStruct((B * L * D,), jnp.float32),
        mesh=mesh,
        scratch_types=scratch,
    )(xf, pf)
    return out.reshape(B, L, D)


# speedup vs baseline: 1.4678x; 1.4678x over previous
import jax
import jax.numpy as jnp
from jax import lax
from jax.experimental import pallas as pl
from jax.experimental.pallas import tpu as pltpu
from jax.experimental.pallas import tpu_sc as plsc

_NC, _NS = 2, 16
_NW = _NC * _NS
_PC = 8
_UNROLL = 8


def _make_sc_kernel(B, L, D):
    lpw = L // _NW
    NP = lpw // _PC
    CW = _PC * D

    mesh = plsc.VectorSubcoreMesh(
        core_axis_name="c", subcore_axis_name="s", num_cores=_NC, num_subcores=_NS
    )

    scratch = (
        [pltpu.VMEM((_PC, D), jnp.float32) for _ in range(8)]
        + [pltpu.VMEM((_PC, D), jnp.float32) for _ in range(2)]
        + [pltpu.SemaphoreType.DMA for _ in range(18)]
    )

    def body(x_hbm, pos_hbm, out_hbm, *scr):
        xb = scr[0:8]
        pb = scr[8:10]
        sx = scr[10:18]
        so = scr[18:26]
        sp = scr[26:28]

        wid = lax.axis_index("s") * _NC + lax.axis_index("c")
        lbase = wid * lpw

        def start_pos(p, par):
            pltpu.async_copy(
                pos_hbm.at[pl.ds(lbase + p * _PC, _PC), :], pb[par], sp[par]
            )

        def wait_pos(par):
            pltpu.make_async_copy(
                pos_hbm.at[pl.ds(0, _PC), :], pb[par], sp[par]
            ).wait()

        def start_x(p, b, par):
            s = par * 4 + b
            pltpu.async_copy(
                x_hbm.at[b, pl.ds(lbase + p * _PC, _PC), :], xb[s], sx[s]
            )

        def wait_x(b, par):
            s = par * 4 + b
            pltpu.make_async_copy(
                x_hbm.at[0, pl.ds(0, _PC), :], xb[s], sx[s]
            ).wait()

        def start_out(p, b, par):
            s = par * 4 + b
            pltpu.async_copy(
                xb[s], out_hbm.at[b, pl.ds(lbase + p * _PC, _PC), :], so[s]
            )

        def wait_out(b, par):
            s = par * 4 + b
            pltpu.make_async_copy(
                xb[s], out_hbm.at[0, pl.ds(0, _PC), :], so[s]
            ).wait()

        def add_chunk(b, par):
            pref = pb[par]
            xref = xb[par * 4 + b]
            span = 16 * _UNROLL
            npc = D // span

            def bf(j, _):
                r = j // npc
                base = (j % npc) * span
                for u in range(_UNROLL):
                    off = base + u * 16
                    plsc.addupdate(
                        xref.at[r, pl.ds(off, 16)], pref[r, pl.ds(off, 16)]
                    )
                return 0

            lax.fori_loop(0, _PC * npc, bf, 0)

        start_pos(0, 0)
        for b in range(B):
            start_x(0, b, 0)

        def loop_body(i, _):
            for par in range(2):
                p = 2 * i + par
                if par == 0:
                    start_pos(p + 1, 1)
                    for b in range(B):
                        @pl.when(i >= 1)
                        def _(b=b):
                            wait_out(b, 1)
                        start_x(p + 1, b, 1)
                else:
                    @pl.when(i < NP // 2 - 1)
                    def _():
                        start_pos(p + 1, 0)
                        for b in range(B):
                            wait_out(b, 0)
                            start_x(p + 1, b, 0)
                wait_pos(par)
                for b in range(B):
                    wait_x(b, par)
                    add_chunk(b, par)
                    start_out(p, b, par)
            return 0

        lax.fori_loop(0, NP // 2, loop_body, 0)

        for b in range(B):
            wait_out(b, 0)
            wait_out(b, 1)

    return mesh, scratch, body


def kernel(x, pos_table):
    B, L, D = x.shape
    mesh, scratch, body = _make_sc_kernel(B, L, D)
    pf = pos_table[:L]
    out = pl.kernel(
        body,
        out_type=jax.ShapeDtypeStruct((B, L, D), jnp.float32),
        mesh=mesh,
        scratch_types=scratch,
    )(x, pf)
    return out


# speedup vs baseline: 3.0724x; 2.0932x over previous
import jax
import jax.numpy as jnp
from jax import lax
from jax.experimental import pallas as pl
from jax.experimental.pallas import tpu as pltpu
from jax.experimental.pallas import tpu_sc as plsc

_NC, _NS = 2, 16
_NW = _NC * _NS
_PC = 8
_UNROLL = 8


def _make_sc_kernel(B, L, D):
    lpw = L // _NW
    NP = lpw // _PC
    CW = _PC * D

    mesh = plsc.VectorSubcoreMesh(
        core_axis_name="c", subcore_axis_name="s", num_cores=_NC, num_subcores=_NS
    )

    scratch = (
        [pltpu.VMEM((_PC, D), jnp.float32) for _ in range(8)]
        + [pltpu.VMEM((_PC, D), jnp.float32) for _ in range(2)]
        + [pltpu.SemaphoreType.DMA for _ in range(18)]
    )

    def body(x_hbm, pos_hbm, out_hbm, *scr):
        xb = scr[0:8]
        pb = scr[8:10]
        sx = scr[10:18]
        so = scr[18:26]
        sp = scr[26:28]

        wid = lax.axis_index("s") * _NC + lax.axis_index("c")
        lbase = wid * lpw

        def start_pos(p, par):
            pltpu.async_copy(
                pos_hbm.at[pl.ds(lbase + p * _PC, _PC), :], pb[par], sp[par]
            )

        def wait_pos(par):
            pltpu.make_async_copy(
                pos_hbm.at[pl.ds(0, _PC), :], pb[par], sp[par]
            ).wait()

        def start_x(p, b, par):
            s = par * 4 + b
            pltpu.async_copy(
                x_hbm.at[b, pl.ds(lbase + p * _PC, _PC), :], xb[s], sx[s]
            )

        def wait_x(b, par):
            s = par * 4 + b
            pltpu.make_async_copy(
                x_hbm.at[0, pl.ds(0, _PC), :], xb[s], sx[s]
            ).wait()

        def start_out(p, b, par):
            s = par * 4 + b
            pltpu.async_copy(
                xb[s], out_hbm.at[b, pl.ds(lbase + p * _PC, _PC), :], so[s]
            )

        def wait_out(b, par):
            s = par * 4 + b
            pltpu.make_async_copy(
                xb[s], out_hbm.at[0, pl.ds(0, _PC), :], so[s]
            ).wait()

        def add_chunk(b, par):
            pref = pb[par]
            xref = xb[par * 4 + b]
            npc = D // 16

            @plsc.parallel_loop(0, npc, unroll=2)
            def _(j):
                off = j * 16
                for r in range(_PC):
                    plsc.addupdate(
                        xref.at[r, pl.ds(off, 16)], pref[r, pl.ds(off, 16)]
                    )

        start_pos(0, 0)
        for b in range(B):
            start_x(0, b, 0)

        def loop_body(i, _):
            for par in range(2):
                p = 2 * i + par
                if par == 0:
                    start_pos(p + 1, 1)
                    for b in range(B):
                        @pl.when(i >= 1)
                        def _(b=b):
                            wait_out(b, 1)
                        start_x(p + 1, b, 1)
                else:
                    @pl.when(i < NP // 2 - 1)
                    def _():
                        start_pos(p + 1, 0)
                        for b in range(B):
                            wait_out(b, 0)
                            start_x(p + 1, b, 0)
                wait_pos(par)
                for b in range(B):
                    wait_x(b, par)
                    add_chunk(b, par)
                    start_out(p, b, par)
            return 0

        lax.fori_loop(0, NP // 2, loop_body, 0)

        for b in range(B):
            wait_out(b, 0)
            wait_out(b, 1)

    return mesh, scratch, body


def kernel(x, pos_table):
    B, L, D = x.shape
    mesh, scratch, body = _make_sc_kernel(B, L, D)
    pf = pos_table[:L]
    out = pl.kernel(
        body,
        out_type=jax.ShapeDtypeStruct((B, L, D), jnp.float32),
        mesh=mesh,
        scratch_types=scratch,
    )(x, pf)
    return out
